# baseline (speedup 1.0000x reference)
import jax
import jax.numpy as jnp
from jax import lax
from jax.experimental import pallas as pl
from jax.experimental.pallas import tpu as pltpu
from jax.experimental.pallas import tpu_sc as plsc

N = 10000
E = 320000
D = 128
DH = D // 2

NC = 2
NS = 16
NW = NC * NS

K = 125
EPT = E // NS
CH = EPT // K
RPT = 624
TAIL = N - RPT * NS
ZR = RPT // 8
LANES = DH // 16


def _spmm_body(g_hbm, dst_hbm, src_hbm, w_hbm, out_hbm,
               acc, msgs0, msgs1, dst_v, src_v, w_v,
               gsem0, gsem1, ssem0, ssem1):
    c = lax.axis_index("c")
    s = lax.axis_index("s")
    row0 = s * CH

    msgs = (msgs0, msgs1)
    gsem = (gsem0, gsem1)
    ssem = (ssem0, ssem1)

    pltpu.sync_copy(dst_hbm.at[pl.ds(row0, CH)], dst_v)
    pltpu.sync_copy(src_hbm.at[pl.ds(row0, CH)], src_v)
    pltpu.sync_copy(w_hbm.at[pl.ds(s * EPT, EPT)], w_v)

    @plsc.parallel_loop(0, K, step=1, unroll=5)
    def _zero_row(r):
        for j in range(LANES):
            msgs0[r, pl.ds(j * 16, 16)] = jnp.zeros((16,), jnp.float32)
    for i in range(RPT // ZR):
        pltpu.sync_copy(msgs0.at[pl.ds(0, ZR)],
                        acc.at[pl.ds(s * RPT + i * ZR, ZR)])

    @pl.when(s == NS - 1)
    def _():
        pltpu.sync_copy(msgs0.at[pl.ds(0, TAIL)],
                        acc.at[pl.ds(RPT * NS, TAIL)])

    plsc.subcore_barrier()

    def _gather(b, g):
        pltpu.async_copy(g_hbm.at[c].at[src_v.at[g]], msgs[b], gsem[b])

    def _gather_wait(b, g):
        pltpu.make_async_copy(g_hbm.at[c].at[src_v.at[g]],
                              msgs[b], gsem[b]).wait()

    def _scatter(b, g):
        pltpu.async_copy(msgs[b], acc.at[dst_v.at[g]], ssem[b], add=True)

    def _scatter_wait(b, g):
        pltpu.make_async_copy(msgs[b], acc.at[dst_v.at[g]], ssem[b]).wait()

    def _scale(b, g):
        m = msgs[b]
        base = g * K

        @plsc.parallel_loop(0, K, step=1, unroll=25)
        def body(k):
            ki = jnp.full((16,), base + k, jnp.int32)
            wk = plsc.load_gather(w_v, [ki])
            for j in range(LANES):
                sl = pl.ds(j * 16, 16)
                m[k, sl] = m[k, sl] * wk

    _gather(0, 0)

    def _chunk_pair(gp, carry):
        for b in (0, 1):
            g = 2 * gp + b
            nb = 1 - b

            @pl.when(g + 1 < CH)
            def _():
                @pl.when(g >= 1)
                def _():
                    _scatter_wait(nb, g - 1)
                _gather(nb, g + 1)

            _gather_wait(b, g)
            _scale(b, g)
            _scatter(b, g)
        return carry

    lax.fori_loop(0, CH // 2, _chunk_pair, None)
    _scatter_wait(0, CH - 2)
    _scatter_wait(1, CH - 1)

    plsc.subcore_barrier()
    pltpu.sync_copy(acc.at[pl.ds(s * RPT, RPT)],
                    out_hbm.at[c, pl.ds(s * RPT, RPT)])

    @pl.when(s == NS - 1)
    def _():
        pltpu.sync_copy(acc.at[pl.ds(RPT * NS, TAIL)],
                        out_hbm.at[c, pl.ds(RPT * NS, TAIL)])


@jax.jit
def _spmm(g2, dst2d, src_flat, w_flat):
    mesh = plsc.VectorSubcoreMesh(core_axis_name="c", subcore_axis_name="s",
                                  num_cores=NC, num_subcores=NS)
    return pl.kernel(
        _spmm_body,
        out_type=jax.ShapeDtypeStruct((NC, N, DH), jnp.float32),
        mesh=mesh,
        compiler_params=pltpu.CompilerParams(needs_layout_passes=False,
                                             use_tc_tiling_on_sc=False),
        scratch_types=[
            pltpu.VMEM_SHARED((N, DH), jnp.float32),
            pltpu.VMEM((K, DH), jnp.float32),
            pltpu.VMEM((K, DH), jnp.float32),
            pltpu.VMEM((CH, K), jnp.int32),
            pltpu.VMEM((CH, K), jnp.int32),
            pltpu.VMEM((EPT,), jnp.float32),
            pltpu.SemaphoreType.DMA,
            pltpu.SemaphoreType.DMA,
            pltpu.SemaphoreType.DMA,
            pltpu.SemaphoreType.DMA,
        ],
    )(g2, dst2d, src_flat, w_flat)


def _transform_body(h_ref, w_ref, g_ref):
    h = h_ref[...]
    for i in range(NC):
        g_ref[i] = lax.dot_general(
            h, w_ref[pl.ds(i * DH, DH), :],
            dimension_numbers=(((1,), (1,)), ((), ())),
            preferred_element_type=jnp.float32)


@jax.jit
def _transform(h, w):
    return pl.pallas_call(
        _transform_body,
        out_shape=jax.ShapeDtypeStruct((NC, N, DH), jnp.float32),
    )(h, w)


def _assemble_body(p_ref, o_ref):
    o_ref[:, pl.ds(0, DH)] = jnp.maximum(p_ref[0], 0.0)
    o_ref[:, pl.ds(DH, DH)] = jnp.maximum(p_ref[1], 0.0)


@jax.jit
def _assemble(partials):
    return pl.pallas_call(
        _assemble_body,
        out_shape=jax.ShapeDtypeStruct((N, D), jnp.float32),
    )(partials)


def kernel(H, edge_index, edge_weight, W):
    dst2d = edge_index[0].reshape(NS * CH, K)
    src_flat = edge_index[1].reshape(NS * CH, K)
    g2 = _transform(H, W)
    partials = _spmm(g2, dst2d, src_flat, edge_weight)
    return _assemble(partials)


# speedup vs baseline: 1.0903x; 1.0903x over previous
import jax
import jax.numpy as jnp
from jax import lax
from jax.experimental import pallas as pl
from jax.experimental.pallas import tpu as pltpu
from jax.experimental.pallas import tpu_sc as plsc

N = 10000
E = 320000
D = 128
DH = D // 2

NC = 2
NS = 16
NW = NC * NS

K = 125
EPT = E // NS
CH = EPT // K
NBUF = 4
KP = 128
RPT = 624
TAIL = N - RPT * NS
ZR = RPT // 8
LANES = DH // 16


def _spmm_body(g_hbm, dst_hbm, src_hbm, w_hbm, out_hbm,
               acc, gbuf0, gbuf1, gbuf2, gbuf3, sbuf0, sbuf1, sbuf2, sbuf3,
               src_v, dring, wring,
               gsem0, gsem1, gsem2, gsem3, ssem0, ssem1, ssem2, ssem3,
               dsem0, dsem1, dsem2, dsem3, wsem0, wsem1, wsem2, wsem3):
    c = lax.axis_index("c")
    s = lax.axis_index("s")
    row0 = s * CH

    gbuf = (gbuf0, gbuf1, gbuf2, gbuf3)
    sbuf = (sbuf0, sbuf1, sbuf2, sbuf3)
    gsem = (gsem0, gsem1, gsem2, gsem3)
    ssem = (ssem0, ssem1, ssem2, ssem3)
    dsem = (dsem0, dsem1, dsem2, dsem3)
    wsem = (wsem0, wsem1, wsem2, wsem3)

    pltpu.sync_copy(src_hbm.at[pl.ds(row0, CH)], src_v)

    @plsc.parallel_loop(0, K, step=1, unroll=5)
    def _zero_row(r):
        for j in range(LANES):
            sbuf0[r, pl.ds(j * 16, 16)] = jnp.zeros((16,), jnp.float32)
    for i in range(RPT // ZR):
        pltpu.sync_copy(sbuf0.at[pl.ds(0, ZR)],
                        acc.at[pl.ds(s * RPT + i * ZR, ZR)])

    @pl.when(s == NS - 1)
    def _():
        pltpu.sync_copy(sbuf0.at[pl.ds(0, TAIL)],
                        acc.at[pl.ds(RPT * NS, TAIL)])

    plsc.subcore_barrier()

    def _gather(b, g):
        pltpu.async_copy(g_hbm.at[c].at[src_v.at[g]], gbuf[b], gsem[b])

    def _gather_wait(b, g):
        pltpu.make_async_copy(g_hbm.at[c].at[src_v.at[g]],
                              gbuf[b], gsem[b]).wait()

    def _dfetch(b, g):
        pltpu.async_copy(dst_hbm.at[row0 + g], dring.at[b], dsem[b])

    def _dfetch_wait(b, g):
        pltpu.make_async_copy(dst_hbm.at[row0 + g],
                              dring.at[b], dsem[b]).wait()

    def _wfetch(b, g):
        pltpu.async_copy(w_hbm.at[row0 + g],
                         wring.at[pl.ds(b * KP, K)], wsem[b])

    def _wfetch_wait(b, g):
        pltpu.make_async_copy(w_hbm.at[row0 + g],
                              wring.at[pl.ds(b * KP, K)], wsem[b]).wait()

    def _scatter(b, g):
        pltpu.async_copy(sbuf[b], acc.at[dring.at[b]], ssem[b], add=True)

    def _scatter_wait(b, g):
        pltpu.make_async_copy(sbuf[b], acc.at[dring.at[b]], ssem[b]).wait()

    def _scale(b):
        gm = gbuf[b]
        sm = sbuf[b]
        base = b * KP

        @plsc.parallel_loop(0, K, step=1, unroll=5)
        def body(k):
            ki = jnp.full((16,), base + k, jnp.int32)
            wk = plsc.load_gather(wring, [ki])
            for j in range(LANES):
                sl = pl.ds(j * 16, 16)
                sm[k, sl] = gm[k, sl] * wk

    for b in range(NBUF):
        _wfetch(b, b)
        _dfetch(b, b)
        _gather(b, b)

    def _chunk_quad(gq, carry):
        for b in range(NBUF):
            g = NBUF * gq + b
            _gather_wait(b, g)
            _wfetch_wait(b, g)

            @pl.when(g >= NBUF)
            def _():
                _scatter_wait(b, g - NBUF)

            _scale(b)
            _dfetch_wait(b, g)
            _scatter(b, g)

            @pl.when(g + NBUF < CH)
            def _():
                _wfetch(b, g + NBUF)
                _dfetch(b, g + NBUF)
                _gather(b, g + NBUF)
        return carry

    lax.fori_loop(0, CH // NBUF, _chunk_quad, None)
    for b in range(NBUF):
        _scatter_wait(b, CH - NBUF + b)

    plsc.subcore_barrier()
    pltpu.sync_copy(acc.at[pl.ds(s * RPT, RPT)],
                    out_hbm.at[c, pl.ds(s * RPT, RPT)])

    @pl.when(s == NS - 1)
    def _():
        pltpu.sync_copy(acc.at[pl.ds(RPT * NS, TAIL)],
                        out_hbm.at[c, pl.ds(RPT * NS, TAIL)])


@jax.jit
def _spmm(g2, dst2d, src2d, w2d):
    mesh = plsc.VectorSubcoreMesh(core_axis_name="c", subcore_axis_name="s",
                                  num_cores=NC, num_subcores=NS)
    return pl.kernel(
        _spmm_body,
        out_type=jax.ShapeDtypeStruct((NC, N, DH), jnp.float32),
        mesh=mesh,
        compiler_params=pltpu.CompilerParams(needs_layout_passes=False,
                                             use_tc_tiling_on_sc=False),
        scratch_types=(
            [pltpu.VMEM_SHARED((N, DH), jnp.float32)]
            + [pltpu.VMEM((K, DH), jnp.float32)] * NBUF
            + [pltpu.VMEM((K, DH), jnp.float32)] * NBUF
            + [pltpu.VMEM((CH, K), jnp.int32)]
            + [pltpu.VMEM((NBUF, K), jnp.int32)]
            + [pltpu.VMEM((NBUF * KP,), jnp.float32)]
            + [pltpu.SemaphoreType.DMA] * (4 * NBUF)
        ),
    )(g2, dst2d, src2d, w2d)


def _transform_body(h_ref, w_ref, g_ref):
    h = h_ref[...]
    for i in range(NC):
        g_ref[i] = lax.dot_general(
            h, w_ref[pl.ds(i * DH, DH), :],
            dimension_numbers=(((1,), (1,)), ((), ())),
            preferred_element_type=jnp.float32)


@jax.jit
def _transform(h, w):
    return pl.pallas_call(
        _transform_body,
        out_shape=jax.ShapeDtypeStruct((NC, N, DH), jnp.float32),
    )(h, w)


def _assemble_body(p_ref, o_ref):
    o_ref[:, pl.ds(0, DH)] = jnp.maximum(p_ref[0], 0.0)
    o_ref[:, pl.ds(DH, DH)] = jnp.maximum(p_ref[1], 0.0)


@jax.jit
def _assemble(partials):
    return pl.pallas_call(
        _assemble_body,
        out_shape=jax.ShapeDtypeStruct((N, D), jnp.float32),
    )(partials)


def kernel(H, edge_index, edge_weight, W):
    dst2d = edge_index[0].reshape(NS * CH, K)
    src2d = edge_index[1].reshape(NS * CH, K)
    w2d = edge_weight.reshape(NS * CH, K)
    g2 = _transform(H, W)
    partials = _spmm(g2, dst2d, src2d, w2d)
    return _assemble(partials)
